# Initial kernel scaffold; baseline (speedup 1.0000x reference)
import functools

import jax
import jax.numpy as jnp
from jax import lax
from jax.experimental import pallas as pl
from jax.experimental.pallas import tpu as pltpu
from jax.experimental.pallas import tpu_sc as plsc

_B, _C, _H, _W = 2, 96, 384, 384
_HW = _H * _W
_NT = 16
_NC = 2
_CB = 8
_NCB = _C // _CB
_PT = _HW // _NT
_CHUNK = 1152
_NCHUNK = _PT // _CHUNK
_G128 = _CHUNK // 128
_G16 = _CHUNK // 16


_HB = 96


def _taps_body(flow_ref, idx_ref, wgt_ref):
    h = pl.program_id(1)
    fx = flow_ref[0, 0]
    fy = flow_ref[0, 1]
    shp = fx.shape
    x = lax.broadcasted_iota(jnp.float32, shp, 1)
    y = lax.broadcasted_iota(jnp.float32, shp, 0)
    y = y + (h * _HB).astype(jnp.float32)
    out_x = x + fx
    out_y = y + fy
    x0f = jnp.floor(out_x)
    y0f = jnp.floor(out_y)
    x0 = x0f.astype(jnp.int32)
    y0 = y0f.astype(jnp.int32)
    ax = out_x - x0f
    ay = out_y - y0f
    bx = 1.0 - ax
    by = 1.0 - ay
    taps = (
        (x0, y0, bx * by),
        (x0 + 1, y0, ax * by),
        (x0, y0 + 1, bx * ay),
        (x0 + 1, y0 + 1, ax * ay),
    )
    for t, (xi, yi, w) in enumerate(taps):
        valid = (xi >= 0) & (xi < _W) & (yi >= 0) & (yi < _H)
        idx_ref[0, t] = jnp.where(valid, yi * _W + xi, 0)
        wgt_ref[0, t] = jnp.where(valid, w, 0.0)


def _taps(flow):
    return pl.pallas_call(
        _taps_body,
        grid=(_B, _H // _HB),
        in_specs=[pl.BlockSpec((1, 2, _HB, _W), lambda b, h: (b, 0, h, 0))],
        out_specs=[
            pl.BlockSpec((1, 4, _HB, _W), lambda b, h: (b, 0, h, 0)),
            pl.BlockSpec((1, 4, _HB, _W), lambda b, h: (b, 0, h, 0)),
        ],
        out_shape=[
            jax.ShapeDtypeStruct((_B, 4, _H, _W), jnp.int32),
            jax.ShapeDtypeStruct((_B, 4, _H, _W), jnp.float32),
        ],
    )(flow)


def _splat_sc_body(frame, idxr, wgt, zeros, out, table, vraw, vals, idxb, wb,
                   zbuf, sem):
    cid = lax.axis_index("c")
    sid = lax.axis_index("s")
    tile_p0 = sid * _PT
    tile_g0 = sid * (_PT // 128)

    pltpu.sync_copy(zeros, zbuf)
    i16 = lax.iota(jnp.int32, 16)
    cols = [jnp.full((16,), c, jnp.int32) for c in range(_CB)]

    for b in range(_B):

        def task_body(i, carry):
            cb = 2 * i + cid

            def zero_loop(k, c):
                pltpu.sync_copy(
                    zbuf, table.at[pl.ds(tile_p0 + k * _CHUNK, _CHUNK), :])
                return c

            lax.fori_loop(0, _NCHUNK, zero_loop, 0)
            plsc.subcore_barrier()

            def chunk_body(k, c):
                pbase = tile_p0 + k * _CHUNK
                pltpu.sync_copy(
                    frame.at[b, pl.ds(cb * _CB, _CB), pl.ds(pbase, _CHUNK)],
                    vraw)
                pltpu.sync_copy(
                    idxr.at[b, :, pl.ds(tile_g0 + k * _G128, _G128), :], idxb)
                pltpu.sync_copy(wgt.at[b, :, pl.ds(pbase, _CHUNK)], wb)

                def cgroup(g, cc):
                    rb = g * 16
                    rowv0 = i16 + rb
                    for t in range(4):
                        wv = wb[t, pl.ds(rb, 16)]
                        rowv = rowv0 + t * _CHUNK
                        for ch in range(_CB):
                            v = vraw[ch, pl.ds(rb, 16)]
                            plsc.store_scatter(vals, [rowv, cols[ch]], v * wv)
                    return cc

                lax.fori_loop(0, _G16, cgroup, 0)

                descs = []
                for t in range(4):
                    for g in range(_G128):
                        descs.append(
                            pltpu.async_copy(
                                vals.at[pl.ds(t * _CHUNK + g * 128, 128), :],
                                table.at[idxb.at[t, g]],
                                sem,
                                add=True))
                for d in descs:
                    d.wait()
                return c

            lax.fori_loop(0, _NCHUNK, chunk_body, 0)
            plsc.subcore_barrier()

            def out_loop(k, c):
                pbase = tile_p0 + k * _CHUNK
                pltpu.sync_copy(
                    table.at[pl.ds(pbase, _CHUNK), :],
                    out.at[b, cb, pl.ds(pbase, _CHUNK), :])
                return c

            lax.fori_loop(0, _NCHUNK, out_loop, 0)
            return carry

        lax.fori_loop(0, _NCB // _NC, task_body, 0)


_splat_sc = functools.partial(
    pl.kernel,
    out_type=jax.ShapeDtypeStruct((_B, _NCB, _HW, _CB), jnp.float32),
    mesh=plsc.VectorSubcoreMesh(core_axis_name="c", subcore_axis_name="s"),
    scratch_types=[
        pltpu.VMEM_SHARED((_HW, _CB), jnp.float32),
        pltpu.VMEM((_CB, _CHUNK), jnp.float32),
        pltpu.VMEM((4 * _CHUNK, _CB), jnp.float32),
        pltpu.VMEM((4, _G128, 128), jnp.int32),
        pltpu.VMEM((4, _CHUNK), jnp.float32),
        pltpu.VMEM((_CHUNK, _CB), jnp.float32),
        pltpu.SemaphoreType.DMA,
    ],
)(_splat_sc_body)


@jax.jit
def kernel(frame, flow):
    idx, wgt = _taps(flow)
    out4 = _splat_sc(
        frame.reshape(_B, _C, _HW),
        idx.reshape(_B, 4, _HW // 128, 128),
        wgt.reshape(_B, 4, _HW),
        jnp.zeros((_CHUNK, _CB), jnp.float32),
    )
    return out4.transpose(0, 1, 3, 2).reshape(_B, _C, _H, _W)


# speedup vs baseline: 2.1883x; 2.1883x over previous
import functools

import jax
import jax.numpy as jnp
from jax import lax
from jax.experimental import pallas as pl
from jax.experimental.pallas import tpu as pltpu
from jax.experimental.pallas import tpu_sc as plsc

_B, _C, _H, _W = 2, 96, 384, 384
_HW = _H * _W
_NT = 16
_NC = 2
_CB = 8
_NCB = _C // _CB
_PT = _HW // _NT
_CHUNK = 1024
_NCHUNK = _PT // _CHUNK
_G128 = _CHUNK // 128
_G16 = _CHUNK // 16


_HB = 96


def _taps_body(flow_ref, idx_ref, wgt_ref):
    h = pl.program_id(1)
    fx = flow_ref[0, 0]
    fy = flow_ref[0, 1]
    shp = fx.shape
    x = lax.broadcasted_iota(jnp.int32, shp, 1).astype(jnp.float32)
    y = lax.broadcasted_iota(jnp.int32, shp, 0).astype(jnp.float32)
    y = y + (h * _HB).astype(jnp.float32)
    out_x = x + fx
    out_y = y + fy
    x0f = jnp.floor(out_x)
    y0f = jnp.floor(out_y)
    x0 = x0f.astype(jnp.int32)
    y0 = y0f.astype(jnp.int32)
    ax = out_x - x0f
    ay = out_y - y0f
    bx = 1.0 - ax
    by = 1.0 - ay
    taps = (
        (x0, y0, bx * by),
        (x0 + 1, y0, ax * by),
        (x0, y0 + 1, bx * ay),
        (x0 + 1, y0 + 1, ax * ay),
    )
    for t, (xi, yi, w) in enumerate(taps):
        valid = (xi >= 0) & (xi < _W) & (yi >= 0) & (yi < _H)
        idx_ref[0, t] = jnp.where(valid, yi * _W + xi, 0)
        wgt_ref[0, t] = jnp.where(valid, w, 0.0)


def _taps(flow):
    return pl.pallas_call(
        _taps_body,
        grid=(_B, _H // _HB),
        in_specs=[pl.BlockSpec((1, 2, _HB, _W), lambda b, h: (b, 0, h, 0))],
        out_specs=[
            pl.BlockSpec((1, 4, _HB, _W), lambda b, h: (b, 0, h, 0)),
            pl.BlockSpec((1, 4, _HB, _W), lambda b, h: (b, 0, h, 0)),
        ],
        out_shape=[
            jax.ShapeDtypeStruct((_B, 4, _H, _W), jnp.int32),
            jax.ShapeDtypeStruct((_B, 4, _H, _W), jnp.float32),
        ],
    )(flow)


def _splat_sc_body(frame, idxr, wgt, zeros, out, table, vraw, wb, idxb, vals,
                   zbuf, sem):
    cid = lax.axis_index("c")
    sid = lax.axis_index("s")
    tile_p0 = sid * _PT
    tile_g0 = sid * (_PT // 128)

    pltpu.sync_copy(zeros, zbuf)
    i16 = lax.iota(jnp.int32, 16)
    cols = [jnp.full((16,), ch, jnp.int32) for ch in range(_CB)]

    for b in range(_B):

        def task_body(i, carry):
            cb = 2 * i + cid

            def zero_loop(k, c):
                pltpu.sync_copy(
                    zbuf, table.at[pl.ds(tile_p0 + k * _CHUNK, _CHUNK), :])
                return c

            lax.fori_loop(0, _NCHUNK, zero_loop, 0)
            plsc.subcore_barrier()

            def chunk_body(k, c):
                pbase = tile_p0 + k * _CHUNK
                pltpu.sync_copy(
                    frame.at[b, pl.ds(cb * _CB, _CB), pl.ds(pbase, _CHUNK)],
                    vraw)
                pltpu.sync_copy(wgt.at[b, :, pl.ds(pbase, _CHUNK)], wb)
                pltpu.sync_copy(
                    idxr.at[b, :, pl.ds(tile_g0 + k * _G128, _G128), :], idxb)

                def cgroup(g, cc):
                    rb = g * 16
                    rowv0 = i16 + rb
                    for t in range(4):
                        wv = wb[t, pl.ds(rb, 16)]
                        rowv = rowv0 + t * _CHUNK
                        for ch in range(_CB):
                            v = vraw[ch, pl.ds(rb, 16)]
                            plsc.store_scatter(vals, [rowv, cols[ch]], v * wv)
                    return cc

                lax.fori_loop(0, _G16, cgroup, 0)

                descs = []
                for t in range(4):
                    for g in range(_G128):
                        descs.append(
                            pltpu.async_copy(
                                vals.at[pl.ds(t * _CHUNK + g * 128, 128), :],
                                table.at[idxb.at[t, g]],
                                sem,
                                add=True))
                for d in descs:
                    d.wait()
                return c

            lax.fori_loop(0, _NCHUNK, chunk_body, 0)
            plsc.subcore_barrier()

            def out_loop(k, c):
                pbase = tile_p0 + k * _CHUNK
                pltpu.sync_copy(
                    table.at[pl.ds(pbase, _CHUNK), :],
                    out.at[b, cb, pl.ds(pbase, _CHUNK), :])
                return c

            lax.fori_loop(0, _NCHUNK, out_loop, 0)
            return carry

        lax.fori_loop(0, _NCB // _NC, task_body, 0)


_splat_sc = functools.partial(
    pl.kernel,
    out_type=jax.ShapeDtypeStruct((_B, _NCB, _HW, _CB), jnp.float32),
    mesh=plsc.VectorSubcoreMesh(core_axis_name="c", subcore_axis_name="s"),
    scratch_types=[
        pltpu.VMEM_SHARED((_HW, _CB), jnp.float32),
        pltpu.VMEM((_CB, _CHUNK), jnp.float32),
        pltpu.VMEM((4, _CHUNK), jnp.float32),
        pltpu.VMEM((4, _G128, 128), jnp.int32),
        pltpu.VMEM((4 * _CHUNK, _CB), jnp.float32),
        pltpu.VMEM((_CHUNK, _CB), jnp.float32),
        pltpu.SemaphoreType.DMA,
    ],
    compiler_params=pltpu.CompilerParams(
        needs_layout_passes=False, use_tc_tiling_on_sc=False),
)(_splat_sc_body)


@jax.jit
def kernel(frame, flow):
    idx, wgt = _taps(flow)
    out4 = _splat_sc(
        frame.reshape(_B, _C, _HW),
        idx.reshape(_B, 4, _HW // 128, 128),
        wgt.reshape(_B, 4, _HW),
        jnp.zeros((_CHUNK, _CB), jnp.float32),
    )
    return out4.transpose(0, 1, 3, 2).reshape(_B, _C, _H, _W)


# speedup vs baseline: 2.2748x; 1.0396x over previous
import functools

import jax
import jax.numpy as jnp
from jax import lax
from jax.experimental import pallas as pl
from jax.experimental.pallas import tpu as pltpu
from jax.experimental.pallas import tpu_sc as plsc

_B, _C, _H, _W = 2, 96, 384, 384
_HW = _H * _W
_NT = 16
_NC = 2
_CB = 8
_NCB = _C // _CB
_PT = _HW // _NT
_CHUNK = 1024
_NCHUNK = _PT // _CHUNK
_G128 = _CHUNK // 128
_G16 = _CHUNK // 16


_HB = 96


def _taps_body(flow_ref, idx_ref, wgt_ref):
    h = pl.program_id(1)
    fx = flow_ref[0, 0]
    fy = flow_ref[0, 1]
    shp = fx.shape
    x = lax.broadcasted_iota(jnp.int32, shp, 1).astype(jnp.float32)
    y = lax.broadcasted_iota(jnp.int32, shp, 0).astype(jnp.float32)
    y = y + (h * _HB).astype(jnp.float32)
    out_x = x + fx
    out_y = y + fy
    x0f = jnp.floor(out_x)
    y0f = jnp.floor(out_y)
    x0 = x0f.astype(jnp.int32)
    y0 = y0f.astype(jnp.int32)
    ax = out_x - x0f
    ay = out_y - y0f
    bx = 1.0 - ax
    by = 1.0 - ay
    taps = (
        (x0, y0, bx * by),
        (x0 + 1, y0, ax * by),
        (x0, y0 + 1, bx * ay),
        (x0 + 1, y0 + 1, ax * ay),
    )
    for t, (xi, yi, w) in enumerate(taps):
        valid = (xi >= 0) & (xi < _W) & (yi >= 0) & (yi < _H)
        idx_ref[0, t] = jnp.where(valid, yi * _W + xi, 0)
        wgt_ref[0, t] = jnp.where(valid, w, 0.0)


def _taps(flow):
    return pl.pallas_call(
        _taps_body,
        grid=(_B, _H // _HB),
        in_specs=[pl.BlockSpec((1, 2, _HB, _W), lambda b, h: (b, 0, h, 0))],
        out_specs=[
            pl.BlockSpec((1, 4, _HB, _W), lambda b, h: (b, 0, h, 0)),
            pl.BlockSpec((1, 4, _HB, _W), lambda b, h: (b, 0, h, 0)),
        ],
        out_shape=[
            jax.ShapeDtypeStruct((_B, 4, _H, _W), jnp.int32),
            jax.ShapeDtypeStruct((_B, 4, _H, _W), jnp.float32),
        ],
    )(flow)


def _splat_sc_body(frame, idxr, wgt, zeros, out, table, vraw_a, vraw_b, wb_a,
                   wb_b, idxb, vals_0, vals_1, sem_in_a, sem_in_b, sem_sc_0,
                   sem_sc_1, sem_misc):
    cid = lax.axis_index("c")
    sid = lax.axis_index("s")
    tile_p0 = sid * _PT
    tile_g0 = sid * (_PT // 128)

    i16 = lax.iota(jnp.int32, 16)
    cols = [jnp.full((16,), ch, jnp.int32) for ch in range(_CB)]
    vraws = (vraw_a, vraw_b)
    wbs = (wb_a, wb_b)
    valss = (vals_0, vals_1)
    sems_in = (sem_in_a, sem_in_b)
    sems_sc = (sem_sc_0, sem_sc_1)

    def task_body(i, carry):
        b = i // (_NCB // _NC)
        cb = 2 * lax.rem(i, _NCB // _NC) + cid

        pre = []
        for k in range(_NCHUNK):
            pre.append(
                pltpu.async_copy(
                    zeros, table.at[pl.ds(tile_p0 + k * _CHUNK, _CHUNK), :],
                    sem_misc))
        for d in pre:
            d.wait()
        plsc.subcore_barrier()

        def issue_inputs(k):
            pbase = tile_p0 + k * _CHUNK
            j = k % 2
            return [
                pltpu.async_copy(
                    frame.at[b, pl.ds(cb * _CB, _CB), pl.ds(pbase, _CHUNK)],
                    vraws[j], sems_in[j]),
                pltpu.async_copy(
                    wgt.at[b, :, pl.ds(pbase, _CHUNK)], wbs[j], sems_in[j]),
                pltpu.async_copy(
                    idxr.at[b, :, pl.ds(tile_g0 + k * _G128, _G128), :],
                    idxb.at[k % 3], sems_in[j]),
            ]

        def compute_tap(k, t):
            j = k % 2
            vraw, wb, vals = vraws[j], wbs[j], valss[t % 2]

            def cgroup(g, cc):
                rb = g * 16
                rowv = i16 + rb
                wv = wb[t, pl.ds(rb, 16)]
                for ch in range(_CB):
                    v = vraw[ch, pl.ds(rb, 16)]
                    plsc.store_scatter(vals, [rowv, cols[ch]], v * wv)
                return cc

            lax.fori_loop(0, _G16, cgroup, 0)

        def fire_tap(k, t):
            descs = []
            for g in range(_G128):
                descs.append(
                    pltpu.async_copy(
                        valss[t % 2].at[pl.ds(g * 128, 128), :],
                        table.at[idxb.at[k % 3, t, g]],
                        sems_sc[t % 2],
                        add=True))
            return descs

        in_descs = {0: issue_inputs(0)}
        sc_descs = {}
        units = [(k, t) for k in range(_NCHUNK) for t in range(4)]
        for k in range(_NCHUNK):
            if k + 1 < _NCHUNK:
                in_descs[k + 1] = issue_inputs(k + 1)
            for d in in_descs.pop(k):
                d.wait()
            for t in range(4):
                u = 4 * k + t
                if u >= 2:
                    for d in sc_descs.pop(units[u - 2]):
                        d.wait()
                compute_tap(k, t)
                sc_descs[(k, t)] = fire_tap(k, t)
        for key in sorted(sc_descs):
            for d in sc_descs[key]:
                d.wait()
        plsc.subcore_barrier()

        outs = []
        for k in range(_NCHUNK):
            pbase = tile_p0 + k * _CHUNK
            outs.append(
                pltpu.async_copy(
                    table.at[pl.ds(pbase, _CHUNK), :],
                    out.at[b, cb, pl.ds(pbase, _CHUNK), :], sem_misc))
        for d in outs:
            d.wait()
        return carry

    lax.fori_loop(0, _B * _NCB // _NC, task_body, 0)


_splat_sc = functools.partial(
    pl.kernel,
    out_type=jax.ShapeDtypeStruct((_B, _NCB, _HW, _CB), jnp.float32),
    mesh=plsc.VectorSubcoreMesh(core_axis_name="c", subcore_axis_name="s"),
    scratch_types=[
        pltpu.VMEM_SHARED((_HW, _CB), jnp.float32),
        pltpu.VMEM((_CB, _CHUNK), jnp.float32),
        pltpu.VMEM((_CB, _CHUNK), jnp.float32),
        pltpu.VMEM((4, _CHUNK), jnp.float32),
        pltpu.VMEM((4, _CHUNK), jnp.float32),
        pltpu.VMEM((3, 4, _G128, 128), jnp.int32),
        pltpu.VMEM((_CHUNK, _CB), jnp.float32),
        pltpu.VMEM((_CHUNK, _CB), jnp.float32),
        pltpu.SemaphoreType.DMA,
        pltpu.SemaphoreType.DMA,
        pltpu.SemaphoreType.DMA,
        pltpu.SemaphoreType.DMA,
        pltpu.SemaphoreType.DMA,
    ],
    compiler_params=pltpu.CompilerParams(
        needs_layout_passes=False, use_tc_tiling_on_sc=False),
)(_splat_sc_body)


@jax.jit
def kernel(frame, flow):
    idx, wgt = _taps(flow)
    out4 = _splat_sc(
        frame.reshape(_B, _C, _HW),
        idx.reshape(_B, 4, _HW // 128, 128),
        wgt.reshape(_B, 4, _HW),
        jnp.zeros((_CHUNK, _CB), jnp.float32),
    )
    return out4.transpose(0, 1, 3, 2).reshape(_B, _C, _H, _W)


# speedup vs baseline: 3.8867x; 1.7086x over previous
import functools

import jax
import jax.numpy as jnp
from jax import lax
from jax.experimental import pallas as pl
from jax.experimental.pallas import tpu as pltpu
from jax.experimental.pallas import tpu_sc as plsc

_B, _C, _H, _W = 2, 96, 384, 384
_HW = _H * _W
_NT = 16
_NC = 2
_CB = 8
_NCB = _C // _CB
_PT = _HW // _NT
_CHUNK = 1024
_NCHUNK = _PT // _CHUNK
_G128 = _CHUNK // 128
_G16 = _CHUNK // 16


_HB = 96


def _taps_body(flow_ref, idx_ref, wgt_ref):
    h = pl.program_id(1)
    fx = flow_ref[0, 0]
    fy = flow_ref[0, 1]
    shp = fx.shape
    x = lax.broadcasted_iota(jnp.int32, shp, 1).astype(jnp.float32)
    y = lax.broadcasted_iota(jnp.int32, shp, 0).astype(jnp.float32)
    y = y + (h * _HB).astype(jnp.float32)
    out_x = x + fx
    out_y = y + fy
    x0f = jnp.floor(out_x)
    y0f = jnp.floor(out_y)
    x0 = x0f.astype(jnp.int32)
    y0 = y0f.astype(jnp.int32)
    ax = out_x - x0f
    ay = out_y - y0f
    bx = 1.0 - ax
    by = 1.0 - ay
    taps = (
        (x0, y0, bx * by),
        (x0 + 1, y0, ax * by),
        (x0, y0 + 1, bx * ay),
        (x0 + 1, y0 + 1, ax * ay),
    )
    for t, (xi, yi, w) in enumerate(taps):
        valid = (xi >= 0) & (xi < _W) & (yi >= 0) & (yi < _H)
        idx_ref[0, t] = jnp.where(valid, yi * _W + xi, 0)
        wgt_ref[0, t] = jnp.where(valid, w, 0.0)


def _taps(flow):
    return pl.pallas_call(
        _taps_body,
        grid=(_B, _H // _HB),
        in_specs=[pl.BlockSpec((1, 2, _HB, _W), lambda b, h: (b, 0, h, 0))],
        out_specs=[
            pl.BlockSpec((1, 4, _HB, _W), lambda b, h: (b, 0, h, 0)),
            pl.BlockSpec((1, 4, _HB, _W), lambda b, h: (b, 0, h, 0)),
        ],
        out_shape=[
            jax.ShapeDtypeStruct((_B, 4, _H, _W), jnp.int32),
            jax.ShapeDtypeStruct((_B, 4, _H, _W), jnp.float32),
        ],
    )(flow)


def _splat_sc_body(frame, idxr, wgt, zeros, out, table, vraw_a, vraw_b, wb_a,
                   wb_b, idxb, vals_0, vals_1, sem_in_a, sem_in_b, sem_sc_0,
                   sem_sc_1, sem_misc):
    cid = lax.axis_index("c")
    sid = lax.axis_index("s")
    tile_p0 = sid * _PT
    tile_g0 = sid * (_PT // 128)

    i16 = lax.iota(jnp.int32, 16)
    cols = [jnp.full((16,), ch, jnp.int32) for ch in range(_CB)]
    vraws = (vraw_a, vraw_b)
    wbs = (wb_a, wb_b)
    valss = (vals_0, vals_1)
    sems_in = (sem_in_a, sem_in_b)
    sems_sc = (sem_sc_0, sem_sc_1)

    def task_body(i, carry):
        b = i // (_NCB // _NC)
        cb = 2 * lax.rem(i, _NCB // _NC) + cid

        pre = []
        for k in range(_NCHUNK):
            pre.append(
                pltpu.async_copy(
                    zeros, table.at[pl.ds(tile_p0 + k * _CHUNK, _CHUNK), :],
                    sem_misc))
        for d in pre:
            d.wait()
        plsc.subcore_barrier()

        def issue_inputs(k):
            pbase = tile_p0 + k * _CHUNK
            j = k % 2
            return [
                pltpu.async_copy(
                    frame.at[b, pl.ds(cb * _CB, _CB), pl.ds(pbase, _CHUNK)],
                    vraws[j], sems_in[j]),
                pltpu.async_copy(
                    wgt.at[b, :, pl.ds(pbase, _CHUNK)], wbs[j], sems_in[j]),
                pltpu.async_copy(
                    idxr.at[b, :, pl.ds(tile_g0 + k * _G128, _G128), :],
                    idxb.at[k % 3], sems_in[j]),
            ]

        def compute_tap(k, t):
            j = k % 2
            vraw, wb, vals = vraws[j], wbs[j], valss[t % 2]

            def cgroup(g, cc):
                rb = g * 16
                rowv = i16 + rb
                wv = wb[t, pl.ds(rb, 16)]
                for ch in range(_CB):
                    v = vraw[ch, pl.ds(rb, 16)]
                    plsc.store_scatter(vals, [rowv, cols[ch]], v * wv)
                return cc

            lax.fori_loop(0, _G16, cgroup, 0)

        def fire_tap(k, t):
            descs = []
            for g in range(_G128):
                descs.append(
                    pltpu.async_copy(
                        valss[t % 2].at[pl.ds(g * 128, 128), :],
                        table.at[idxb.at[k % 3, t, g]],
                        sems_sc[t % 2],
                        add=True))
            return descs

        in_descs = {0: issue_inputs(0)}
        sc_descs = {}
        units = [(k, t) for k in range(_NCHUNK) for t in range(4)]
        for k in range(_NCHUNK):
            if k + 1 < _NCHUNK:
                in_descs[k + 1] = issue_inputs(k + 1)
            for d in in_descs.pop(k):
                d.wait()
            for t in range(4):
                u = 4 * k + t
                if u >= 2:
                    for d in sc_descs.pop(units[u - 2]):
                        d.wait()
                compute_tap(k, t)
                sc_descs[(k, t)] = fire_tap(k, t)
        for key in sorted(sc_descs):
            for d in sc_descs[key]:
                d.wait()
        plsc.subcore_barrier()

        in_d = {0: pltpu.async_copy(
            table.at[pl.ds(tile_p0, _CHUNK), :], valss[0], sems_in[0])}
        out_d = {}
        for k in range(_NCHUNK):
            j = k % 2
            pbase = tile_p0 + k * _CHUNK
            if k + 1 < _NCHUNK:
                in_d[k + 1] = pltpu.async_copy(
                    table.at[pl.ds(pbase + _CHUNK, _CHUNK), :],
                    valss[(k + 1) % 2], sems_in[(k + 1) % 2])
            in_d.pop(k).wait()
            if k >= 2:
                out_d.pop(k - 2).wait()
            obuf, obuft = valss[j], vraws[j]

            def trans(g, cc):
                rows = i16 + g * 16
                for ch in range(_CB):
                    v = plsc.load_gather(obuf, [rows, cols[ch]])
                    obuft[ch, pl.ds(g * 16, 16)] = v
                return cc

            lax.fori_loop(0, _G16, trans, 0)
            out_d[k] = pltpu.async_copy(
                obuft,
                out.at[b, pl.ds(cb * _CB, _CB), pl.ds(pbase, _CHUNK)],
                sems_sc[j])
        for k in sorted(out_d):
            out_d[k].wait()
        return carry

    lax.fori_loop(0, _B * _NCB // _NC, task_body, 0)


_splat_sc = functools.partial(
    pl.kernel,
    out_type=jax.ShapeDtypeStruct((_B, _C, _HW), jnp.float32),
    mesh=plsc.VectorSubcoreMesh(core_axis_name="c", subcore_axis_name="s"),
    scratch_types=[
        pltpu.VMEM_SHARED((_HW, _CB), jnp.float32),
        pltpu.VMEM((_CB, _CHUNK), jnp.float32),
        pltpu.VMEM((_CB, _CHUNK), jnp.float32),
        pltpu.VMEM((4, _CHUNK), jnp.float32),
        pltpu.VMEM((4, _CHUNK), jnp.float32),
        pltpu.VMEM((3, 4, _G128, 128), jnp.int32),
        pltpu.VMEM((_CHUNK, _CB), jnp.float32),
        pltpu.VMEM((_CHUNK, _CB), jnp.float32),
        pltpu.SemaphoreType.DMA,
        pltpu.SemaphoreType.DMA,
        pltpu.SemaphoreType.DMA,
        pltpu.SemaphoreType.DMA,
        pltpu.SemaphoreType.DMA,
    ],
    compiler_params=pltpu.CompilerParams(
        needs_layout_passes=False, use_tc_tiling_on_sc=False),
)(_splat_sc_body)


@jax.jit
def kernel(frame, flow):
    idx, wgt = _taps(flow)
    out3 = _splat_sc(
        frame.reshape(_B, _C, _HW),
        idx.reshape(_B, 4, _HW // 128, 128),
        wgt.reshape(_B, 4, _HW),
        jnp.zeros((_CHUNK, _CB), jnp.float32),
    )
    return out3.reshape(_B, _C, _H, _W)
